# baseline (speedup 1.0000x reference)
import math

import jax
import jax.numpy as jnp
from jax import lax
from jax.experimental import pallas as pl
from jax.experimental.pallas import tpu as pltpu
from jax.experimental.pallas import tpu_sc as plsc

N = 10000
E = 160000
C = 192
Q = 64
B = 128
NCHUNK = E // B
NS = 16
NC = 2
INV_SQRT2 = 0.7071067811865476
F32 = jnp.float32

_mesh = plsc.VectorSubcoreMesh(core_axis_name="c", subcore_axis_name="s")


def _prep_body(px, py, pz, send, recv, z640, shst, degf, *sc):
    (si0, ri0, si1, ri1, si2, ri2,
     c00, c01, c02, c03, c04, c05,
     c10, c11, c12, c13, c14, c15,
     c20, c21, c22, c23, c24, c25,
     t0, t1, t2, onesb, zb, degs,
     i0, i1, i2, p0, p1, p2, w0, w1, w2) = sc
    sidx = (si0, si1, si2)
    ridx = (ri0, ri1, ri2)
    comp = ((c00, c01, c02, c03, c04, c05),
            (c10, c11, c12, c13, c14, c15),
            (c20, c21, c22, c23, c24, c25))
    shT = (t0, t1, t2)
    semI = (i0, i1, i2)
    semP = (p0, p1, p2)
    semW = (w0, w1, w2)
    c = lax.axis_index("c")
    s = lax.axis_index("s")
    wid = s * NC + c

    pltpu.sync_copy(z640, zb)
    o640 = pl.multiple_of(640 * s, 128)

    @pl.when(s < 15)
    def _zmain():
        pltpu.sync_copy(zb, degs.at[pl.ds(o640, 640)])

    @pl.when(s == 15)
    def _ztail():
        pltpu.sync_copy(zb.at[pl.ds(0, 400)], degs.at[pl.ds(9600, 400)])

    for g in range(B // 16):
        onesb[pl.ds(g * 16, 16)] = jnp.full((16,), 1.0, F32)
    plsc.subcore_barrier()

    def e0_of(i):
        return (wid + 32 * i) * B

    def fire_idx(i, k):
        e0 = e0_of(i)
        pltpu.async_copy(send.at[pl.ds(e0, B)], sidx[k], semI[k])
        pltpu.async_copy(recv.at[pl.ds(e0, B)], ridx[k], semI[k])

    def wait_idx(i, k):
        e0 = e0_of(i)
        pltpu.make_async_copy(send.at[pl.ds(e0, B)], sidx[k], semI[k]).wait()
        pltpu.make_async_copy(recv.at[pl.ds(e0, B)], ridx[k], semI[k]).wait()

    def fire_gathers(k):
        cb = comp[k]
        pltpu.async_copy(px.at[sidx[k]], cb[0], semP[k])
        pltpu.async_copy(py.at[sidx[k]], cb[1], semP[k])
        pltpu.async_copy(pz.at[sidx[k]], cb[2], semP[k])
        pltpu.async_copy(px.at[ridx[k]], cb[3], semP[k])
        pltpu.async_copy(py.at[ridx[k]], cb[4], semP[k])
        pltpu.async_copy(pz.at[ridx[k]], cb[5], semP[k])

    def wait_gathers(k):
        cb = comp[k]
        pltpu.make_async_copy(px.at[sidx[k]], cb[0], semP[k]).wait()
        pltpu.make_async_copy(py.at[sidx[k]], cb[1], semP[k]).wait()
        pltpu.make_async_copy(pz.at[sidx[k]], cb[2], semP[k]).wait()
        pltpu.make_async_copy(px.at[ridx[k]], cb[3], semP[k]).wait()
        pltpu.make_async_copy(py.at[ridx[k]], cb[4], semP[k]).wait()
        pltpu.make_async_copy(pz.at[ridx[k]], cb[5], semP[k]).wait()

    def compute(i, k):
        wait_gathers(k)
        pltpu.make_async_copy(shT[k], shst.at[pl.ds(0, 16 * B)], semW[k]).wait()
        sxb, syb, szb, rxb, ryb, rzb = comp[k]
        tk = shT[k]
        s3 = math.sqrt(3.0)
        s5 = math.sqrt(5.0)
        s15 = math.sqrt(15.0)
        c1 = 0.25 * math.sqrt(70.0)
        c2 = math.sqrt(105.0)
        c3 = 0.25 * math.sqrt(42.0)
        c4 = 0.5 * math.sqrt(7.0)
        c5 = 0.5 * math.sqrt(105.0)
        for g in range(B // 16):
            sl = pl.ds(g * 16, 16)
            x = rxb[sl] - sxb[sl]
            y = ryb[sl] - syb[sl]
            z = rzb[sl] - szb[sl]
            x2 = x * x
            y2 = y * y
            z2c = z * z
            n2 = x2 + y2 + z2c
            shs = [
                s3 * x, s3 * y, s3 * z,
                s15 * x * y, s15 * y * z, 0.5 * s5 * (3.0 * z2c - n2),
                s15 * x * z, 0.5 * s15 * (x2 - y2),
                c1 * y * (3.0 * x2 - y2), c2 * x * y * z,
                c3 * y * (5.0 * z2c - n2), c4 * z * (5.0 * z2c - 3.0 * n2),
                c3 * x * (5.0 * z2c - n2), c5 * z * (x2 - y2),
                c1 * x * (x2 - 3.0 * y2),
                n2,
            ]
            for k2 in range(16):
                tk[pl.ds(k2 * B + g * 16, 16)] = shs[k2]
        e0 = e0_of(i)
        pltpu.async_copy(tk, shst.at[pl.ds(e0 * 16, 16 * B)], semW[k])
        pltpu.sync_copy(onesb, degs.at[ridx[k]], add=True)

    fire_idx(0, 0)
    fire_idx(1, 1)
    for k in range(3):
        pltpu.async_copy(shT[k], shst.at[pl.ds(e0_of(k) * 16, 16 * B)],
                         semW[k])
    wait_idx(0, 0)
    fire_gathers(0)
    fire_idx(2, 2)
    wait_idx(1, 1)
    fire_gathers(1)
    compute(0, 0)
    fire_idx(3, 0)
    wait_idx(2, 2)
    fire_gathers(2)
    compute(1, 1)
    fire_idx(4, 1)

    def outer2(jj, carry):
        for k in range(3):
            i = 3 + 3 * jj + k
            m = (k + 2) % 3
            wait_idx(i, k)
            fire_gathers(k)
            compute(i - 1, m)

            @pl.when(i + 2 < 39)
            def _fi():
                fire_idx(i + 2, m)

        return carry

    lax.fori_loop(0, 12, outer2, 0)
    compute(38, 38 % 3)
    for k in range(3):
        pltpu.make_async_copy(shT[k], shst.at[pl.ds(0, 16 * B)], semW[k]).wait()

    @pl.when(wid < 2)
    def _tailc():
        e0 = (1248 + wid) * B
        pltpu.async_copy(send.at[pl.ds(e0, B)], sidx[0], semI[0])
        pltpu.async_copy(recv.at[pl.ds(e0, B)], ridx[0], semI[0])
        pltpu.make_async_copy(send.at[pl.ds(e0, B)], sidx[0], semI[0]).wait()
        pltpu.make_async_copy(recv.at[pl.ds(e0, B)], ridx[0], semI[0]).wait()
        fire_gathers(0)
        wait_gathers(0)
        sxb, syb, szb, rxb, ryb, rzb = comp[0]
        tk = shT[0]
        s3 = math.sqrt(3.0)
        s5 = math.sqrt(5.0)
        s15 = math.sqrt(15.0)
        c1 = 0.25 * math.sqrt(70.0)
        c2 = math.sqrt(105.0)
        c3 = 0.25 * math.sqrt(42.0)
        c4 = 0.5 * math.sqrt(7.0)
        c5 = 0.5 * math.sqrt(105.0)
        for g in range(B // 16):
            sl = pl.ds(g * 16, 16)
            x = rxb[sl] - sxb[sl]
            y = ryb[sl] - syb[sl]
            z = rzb[sl] - szb[sl]
            x2 = x * x
            y2 = y * y
            z2c = z * z
            n2 = x2 + y2 + z2c
            shs = [
                s3 * x, s3 * y, s3 * z,
                s15 * x * y, s15 * y * z, 0.5 * s5 * (3.0 * z2c - n2),
                s15 * x * z, 0.5 * s15 * (x2 - y2),
                c1 * y * (3.0 * x2 - y2), c2 * x * y * z,
                c3 * y * (5.0 * z2c - n2), c4 * z * (5.0 * z2c - 3.0 * n2),
                c3 * x * (5.0 * z2c - n2), c5 * z * (x2 - y2),
                c1 * x * (x2 - 3.0 * y2),
                n2,
            ]
            for k2 in range(16):
                tk[pl.ds(k2 * B + g * 16, 16)] = shs[k2]
        pltpu.sync_copy(tk, shst.at[pl.ds(e0 * 16, 16 * B)])
        pltpu.sync_copy(onesb, degs.at[ridx[0]], add=True)

    plsc.subcore_barrier()

    @pl.when(s < 15)
    def _wmain():
        pltpu.sync_copy(degs.at[pl.ds(o640, 640)], zb)
        pltpu.sync_copy(zb, degf.at[pl.ds(c * N + 640 * s, 640)])

    @pl.when(s == 15)
    def _wtail():
        pltpu.sync_copy(degs.at[pl.ds(9600, 400)], zb.at[pl.ds(0, 400)])
        pltpu.sync_copy(zb.at[pl.ds(0, 400)], degf.at[pl.ds(c * N + 9600, 400)])


_prep = pl.kernel(
    _prep_body,
    out_type=(
        jax.ShapeDtypeStruct((NCHUNK * 16 * B,), F32),
        jax.ShapeDtypeStruct((2 * N,), F32),
    ),
    mesh=_mesh,
    scratch_types=(
        [pltpu.VMEM((B,), jnp.int32) for _ in range(6)]
        + [pltpu.VMEM((B,), F32) for _ in range(18)]
        + [pltpu.VMEM((16 * B,), F32) for _ in range(3)]
        + [pltpu.VMEM((B,), F32), pltpu.VMEM((640,), F32),
           pltpu.VMEM_SHARED((N,), F32)]
        + [pltpu.SemaphoreType.DMA for _ in range(9)]
    ),
)


def _scal_body(s1e, recv, z640, t1f, ridx, s1b, zb, tacc, sem):
    c = lax.axis_index("c")
    s = lax.axis_index("s")
    wid = s * NC + c

    pltpu.sync_copy(z640, zb)
    o640 = pl.multiple_of(640 * s, 128)

    @pl.when(s < 15)
    def _zmain():
        pltpu.sync_copy(zb, tacc.at[pl.ds(o640, 640)])

    @pl.when(s == 15)
    def _ztail():
        pltpu.sync_copy(zb.at[pl.ds(0, 400)], tacc.at[pl.ds(9600, 400)])

    plsc.subcore_barrier()
    nloc = 39 + jnp.where(wid < 2, 1, 0)

    def chunk_body(i, carry):
        chunk = wid + 32 * i
        e0 = chunk * B
        d1 = pltpu.async_copy(recv.at[pl.ds(e0, B)], ridx, sem)
        d2 = pltpu.async_copy(s1e.at[pl.ds(e0, B)], s1b, sem)
        d1.wait()
        d2.wait()
        pltpu.sync_copy(s1b, tacc.at[ridx], add=True)
        return carry

    lax.fori_loop(0, nloc, chunk_body, 0)
    plsc.subcore_barrier()

    @pl.when(s < 15)
    def _wmain():
        pltpu.sync_copy(tacc.at[pl.ds(o640, 640)], zb)
        pltpu.sync_copy(zb, t1f.at[pl.ds(c * N + 640 * s, 640)])

    @pl.when(s == 15)
    def _wtail():
        pltpu.sync_copy(tacc.at[pl.ds(9600, 400)], zb.at[pl.ds(0, 400)])
        pltpu.sync_copy(zb.at[pl.ds(0, 400)], t1f.at[pl.ds(c * N + 9600, 400)])


_scal_scatter = pl.kernel(
    _scal_body,
    out_type=jax.ShapeDtypeStruct((2 * N,), F32),
    mesh=_mesh,
    scratch_types=(
        pltpu.VMEM((B,), jnp.int32),
        pltpu.VMEM((B,), F32),
        pltpu.VMEM((640,), F32),
        pltpu.VMEM_SHARED((N,), F32),
        pltpu.SemaphoreType.DMA,
    ),
)


def _zero_acc(zb, aggs, s):
    r0 = pl.multiple_of(632 * s, 8)

    @pl.when(s < 15)
    def _zmain():
        for i in range(4):
            pltpu.sync_copy(zb, aggs.at[pl.ds(r0 + i * 128, 128)])
        pltpu.sync_copy(zb.at[pl.ds(0, 120)], aggs.at[pl.ds(r0 + 512, 120)])

    @pl.when(s == 15)
    def _ztail():
        for i in range(4):
            pltpu.sync_copy(zb, aggs.at[pl.ds(9480 + i * 128, 128)])
        pltpu.sync_copy(zb.at[pl.ds(0, 8)], aggs.at[pl.ds(9992, 8)])


def _write_acc(zb, aggs, aggf, s, ob):
    r0 = pl.multiple_of(632 * s, 8)

    @pl.when(s < 15)
    def _wmain():
        for i in range(4):
            pltpu.sync_copy(aggs.at[pl.ds(r0 + i * 128, 128)], zb)
            pltpu.sync_copy(zb, aggf.at[pl.ds(ob + 632 * s + i * 128, 128)])
        pltpu.sync_copy(aggs.at[pl.ds(r0 + 512, 120)], zb.at[pl.ds(0, 120)])
        pltpu.sync_copy(zb.at[pl.ds(0, 120)],
                        aggf.at[pl.ds(ob + 632 * s + 512, 120)])

    @pl.when(s == 15)
    def _wtail():
        for i in range(4):
            pltpu.sync_copy(aggs.at[pl.ds(9480 + i * 128, 128)], zb)
            pltpu.sync_copy(zb, aggf.at[pl.ds(ob + 9480 + i * 128, 128)])
        pltpu.sync_copy(aggs.at[pl.ds(9992, 8)], zb.at[pl.ds(0, 8)])
        pltpu.sync_copy(zb.at[pl.ds(0, 8)], aggf.at[pl.ds(ob + 9992, 8)])


def _agg_body(tab, s_st, send, recv, z64, aggf, *sc):
    (is0, ir0, is1, ir1, is2, ir2, is3, ir3, is4, ir4, is5, ir5,
     g0, sb0, g1, sb1, g2, sb2, zb, aggs,
     i0, i1, i2, i3, i4, i5,
     sg0, sg1, sg2, ss0, ss1, ss2, sb0s, sb1s, sb2s) = sc
    isx = (is0, is1, is2, is3, is4, is5)
    irx = (ir0, ir1, ir2, ir3, ir4, ir5)
    gbuf = (g0, g1, g2)
    sbuf = (sb0, sb1, sb2)
    semI = (i0, i1, i2, i3, i4, i5)
    semG = (sg0, sg1, sg2)
    semS = (ss0, ss1, ss2)
    semB = (sb0s, sb1s, sb2s)
    c = lax.axis_index("c")
    s = lax.axis_index("s")

    def run_pass(p):
        smode = (0, 1, 2)[p]
        j = 2 * p + c
        off = j * N
        soff = (j - 3) * E
        pltpu.sync_copy(z64, zb)
        _zero_acc(zb, aggs, s)
        plsc.subcore_barrier()

        def e0_of(i):
            return (s + NS * i) * B

        def fire_idx(i, q):
            e0 = e0_of(i)
            pltpu.async_copy(send.at[pl.ds(e0, B)], isx[q], semI[q])
            pltpu.async_copy(recv.at[pl.ds(e0, B)], irx[q], semI[q])

        def wait_idx(i, q):
            e0 = e0_of(i)
            pltpu.make_async_copy(send.at[pl.ds(e0, B)], isx[q], semI[q]).wait()
            pltpu.make_async_copy(recv.at[pl.ds(e0, B)], irx[q], semI[q]).wait()

        def fire_sb(b, e0):
            if smode == 0:
                return
            if smode == 1:
                @pl.when(c == 1)
                def _fs():
                    pltpu.async_copy(s_st.at[pl.ds(soff + e0, B)], sbuf[b],
                                     semB[b])
            else:
                pltpu.async_copy(s_st.at[pl.ds(soff + e0, B)], sbuf[b],
                                 semB[b])

        def mult(b):
            gb = gbuf[b]
            sb = sbuf[b]

            def row_body(r4, rc):
                for u in range(4):
                    r = r4 * 4 + u
                    for k in range(Q // 16):
                        gb[r, pl.ds(16 * k, 16)] = (gb[r, pl.ds(16 * k, 16)]
                                                    * sb[r, pl.ds(16 * k, 16)])
                return rc

            lax.fori_loop(0, B // 4, row_body, 0)

        def wait_mult(b):
            if smode == 0:
                return
            if smode == 1:
                @pl.when(c == 1)
                def _m():
                    pltpu.make_async_copy(s_st.at[pl.ds(0, B)], sbuf[b],
                                          semB[b]).wait()
                    mult(b)
            else:
                pltpu.make_async_copy(s_st.at[pl.ds(0, B)], sbuf[b],
                                      semB[b]).wait()
                mult(b)

        def fire_gather(i, b, q):
            wait_idx(i, q)
            for g in range(B // 16):
                sl = pl.ds(g * 16, 16)
                isx[q][sl] = isx[q][sl] + off
            pltpu.async_copy(tab.at[isx[q]], gbuf[b], semG[b])
            fire_sb(b, e0_of(i))

        def fire_b(b, q):
            pltpu.make_async_copy(tab.at[isx[q]], gbuf[b], semG[b]).wait()
            wait_mult(b)
            pltpu.async_copy(gbuf[b], aggs.at[irx[q]], semS[b], add=True)

        def wait_s(b, q):
            pltpu.make_async_copy(gbuf[b], aggs.at[irx[q]], semS[b]).wait()

        fire_idx(0, 0)
        fire_idx(1, 1)
        fire_idx(2, 2)

        def outer(jj, carry):
            for k in range(6):
                i = 6 * jj + k
                b = k % 3

                @pl.when(i >= 3)
                def _ws():
                    wait_s(b, (k + 3) % 6)

                @pl.when(i + 3 < 78)
                def _fi():
                    fire_idx(i + 3, (k + 3) % 6)

                fire_gather(i, b, k)

                @pl.when(i >= 2)
                def _fb():
                    fire_b((b + 1) % 3, (k + 4) % 6)

            return carry

        lax.fori_loop(0, 13, outer, 0)
        fire_b(1, 4)
        fire_b(2, 5)
        wait_s(0, 3)
        wait_s(1, 4)
        wait_s(2, 5)

        @pl.when(s < 2)
        def _tail():
            e0 = (1248 + s) * B
            d1 = pltpu.async_copy(send.at[pl.ds(e0, B)], isx[0], semI[0])
            d2 = pltpu.async_copy(recv.at[pl.ds(e0, B)], irx[0], semI[0])
            fire_sb(0, e0)
            d1.wait()
            d2.wait()
            for g in range(B // 16):
                sl = pl.ds(g * 16, 16)
                isx[0][sl] = isx[0][sl] + off
            pltpu.async_copy(tab.at[isx[0]], gbuf[0], semG[0]).wait()
            if smode == 1:
                @pl.when(c == 1)
                def _m():
                    pltpu.make_async_copy(s_st.at[pl.ds(0, B)], sbuf[0],
                                          semB[0]).wait()
                    mult(0)
            elif smode == 2:
                pltpu.make_async_copy(s_st.at[pl.ds(0, B)], sbuf[0],
                                      semB[0]).wait()
                mult(0)
            pltpu.sync_copy(gbuf[0], aggs.at[irx[0]], add=True)

        plsc.subcore_barrier()
        _write_acc(zb, aggs, aggf, s, j * N)
        plsc.subcore_barrier()

    run_pass(0)
    run_pass(1)
    run_pass(2)


_agg_layer = pl.kernel(
    _agg_body,
    out_type=jax.ShapeDtypeStruct((6 * N, Q), F32),
    mesh=_mesh,
    compiler_params=pltpu.CompilerParams(use_tc_tiling_on_sc=False),
    scratch_types=(
        [pltpu.VMEM((B,), jnp.int32) for _ in range(12)]
        + [pltpu.VMEM((B, Q), F32) for _ in range(6)]
        + [pltpu.VMEM((B, Q), F32), pltpu.VMEM_SHARED((N, Q), F32)]
        + [pltpu.SemaphoreType.DMA for _ in range(15)]
    ),
)


def _gelu(x):
    return 0.5 * x * (1.0 + jnp.tanh(0.7978845608028654 * (x + 0.044715 * x * x * x)))


def _act12(h):
    lane = lax.broadcasted_iota(jnp.int32, h.shape, 1)
    return jnp.where(lane < 32, _gelu(h), jnp.where(lane < 64, jnp.tanh(h), h))


_CB = 25
_C0 = ((0,), (0,)), ((), ())


def _s_norm(blk):
    n2 = blk[15:16, :]
    r = jnp.sqrt(n2)
    rinv = 1.0 / (r + 1e-8)
    rinv2 = rinv * rinv
    rinv3 = rinv2 * rinv
    fac = jnp.concatenate([
        jnp.broadcast_to(rinv, (3, B)),
        jnp.broadcast_to(rinv2, (5, B)),
        jnp.broadcast_to(rinv3, (7, B)),
    ], axis=0)
    shn = blk[:15, :] * fac
    fix5 = jnp.where(n2 == 0.0, -0.5 * math.sqrt(5.0), shn[5:6, :])
    return jnp.concatenate([shn[:5], fix5, shn[6:]], axis=0)


def _s12_body(sh_ref, w21_ref, w22_ref, s1_ref, s2_ref):
    w21 = w21_ref[...]
    w22 = w22_ref[...]
    for j in range(_CB):
        shn = _s_norm(sh_ref[j])
        sl = pl.ds(j * B, B)
        s1_ref[0, j, :] = lax.dot_general(w21, shn, _C0,
                                          preferred_element_type=F32).reshape(B)
        r2 = lax.dot_general(shn, w22, _C0, preferred_element_type=F32)
        for t in range(3):
            s2_ref[t, sl, :] = r2[:, t * Q:(t + 1) * Q]


def _s12_compute(shst, w21, w22):
    return pl.pallas_call(
        _s12_body,
        grid=(NCHUNK // _CB,),
        in_specs=[
            pl.BlockSpec((_CB, 16, B), lambda g: (g, 0, 0)),
            pl.BlockSpec((15, 1), lambda g: (0, 0)),
            pl.BlockSpec((15, C), lambda g: (0, 0)),
        ],
        out_specs=[
            pl.BlockSpec((1, _CB, B), lambda g: (g, 0, 0)),
            pl.BlockSpec((3, _CB * B, Q), lambda g: (0, g, 0)),
        ],
        out_shape=[
            jax.ShapeDtypeStruct((NCHUNK // _CB, _CB, B), F32),
            jax.ShapeDtypeStruct((3, E, Q), F32),
        ],
    )(shst, w21, w22)


def _s3_body(sh_ref, w23_ref, s3_ref):
    w23 = w23_ref[...]
    for j in range(_CB):
        shn = _s_norm(sh_ref[j])
        sl = pl.ds(j * B, B)
        r3 = lax.dot_general(shn, w23, _C0, preferred_element_type=F32)
        for t in range(3):
            s3_ref[t, sl, :] = r3[:, t * Q:(t + 1) * Q]


def _s3_compute(shst, w23):
    return pl.pallas_call(
        _s3_body,
        grid=(NCHUNK // _CB,),
        in_specs=[
            pl.BlockSpec((_CB, 16, B), lambda g: (g, 0, 0)),
            pl.BlockSpec((15, C), lambda g: (0, 0)),
        ],
        out_specs=pl.BlockSpec((3, _CB * B, Q), lambda g: (0, g, 0)),
        out_shape=jax.ShapeDtypeStruct((3, E, Q), F32),
    )(shst, w23)


_NB = 2000


def _tab_write(tab_ref, x, xb):
    for t in range(3):
        tab_ref[t] = x[:, t * Q:(t + 1) * Q]
        tab_ref[3 + t] = xb[:, t * Q:(t + 1) * Q]


def _l1_body(deg_ref, t1_ref, we11, wpre, wpost, wsc, we1n, tab_ref, x2_ref):
    a0 = deg_ref[0] + deg_ref[1]
    t0 = (t1_ref[0] + t1_ref[1]) * we11[...]
    acat = jnp.concatenate([a0, t0], axis=1) * INV_SQRT2
    h = jnp.dot(acat, wpre[...], preferred_element_type=F32)
    h = _act12(h)
    x2 = wsc[...] + jnp.dot(h, wpost[...], preferred_element_type=F32)
    xb = jnp.dot(x2, we1n[...], preferred_element_type=F32)
    _tab_write(tab_ref, x2, xb)
    x2_ref[...] = x2


def _l1_epilogue(deg2, t12, we11, wpre, wpost, wsc, we1n):
    return pl.pallas_call(
        _l1_body,
        grid=(N // _NB,),
        in_specs=[
            pl.BlockSpec((2, _NB, 1), lambda g: (0, g, 0)),
            pl.BlockSpec((2, _NB, 1), lambda g: (0, g, 0)),
            pl.BlockSpec((1, 1), lambda g: (0, 0)),
            pl.BlockSpec((2, C), lambda g: (0, 0)),
            pl.BlockSpec((C, C), lambda g: (0, 0)),
            pl.BlockSpec((1, C), lambda g: (0, 0)),
            pl.BlockSpec((C, C), lambda g: (0, 0)),
        ],
        out_specs=[
            pl.BlockSpec((6, _NB, Q), lambda g: (0, g, 0)),
            pl.BlockSpec((_NB, C), lambda g: (g, 0)),
        ],
        out_shape=[
            jax.ShapeDtypeStruct((6, N, Q), F32),
            jax.ShapeDtypeStruct((N, C), F32),
        ],
    )(deg2, t12, we11, wpre, wpost, wsc, we1n)


def _merge_h(agg_ref, wpre_ref):
    acat = jnp.concatenate([agg_ref[j] for j in range(6)], axis=1)
    return jnp.dot(acat, wpre_ref[...], preferred_element_type=F32) * INV_SQRT2


def _mid_body(agg_ref, x_ref, wpre, wsc, wpost, we1n, tab_ref, xn_ref):
    h = _act12(_merge_h(agg_ref, wpre))
    xn = (jnp.dot(x_ref[...], wsc[...], preferred_element_type=F32)
          + jnp.dot(h, wpost[...], preferred_element_type=F32))
    xb = jnp.dot(xn, we1n[...], preferred_element_type=F32)
    _tab_write(tab_ref, xn, xb)
    xn_ref[...] = xn


def _mid_epilogue(agg, x, wpre, wsc, wpost, we1n):
    return pl.pallas_call(
        _mid_body,
        grid=(N // _NB,),
        in_specs=[
            pl.BlockSpec((6, _NB, Q), lambda g: (0, g, 0)),
            pl.BlockSpec((_NB, C), lambda g: (g, 0)),
            pl.BlockSpec((2 * C, C), lambda g: (0, 0)),
            pl.BlockSpec((C, C), lambda g: (0, 0)),
            pl.BlockSpec((C, C), lambda g: (0, 0)),
            pl.BlockSpec((C, C), lambda g: (0, 0)),
        ],
        out_specs=[
            pl.BlockSpec((6, _NB, Q), lambda g: (0, g, 0)),
            pl.BlockSpec((_NB, C), lambda g: (g, 0)),
        ],
        out_shape=[
            jax.ShapeDtypeStruct((6, N, Q), F32),
            jax.ShapeDtypeStruct((N, C), F32),
        ],
    )(agg, x, wpre, wsc, wpost, we1n)


def _final_body(agg_ref, x_ref, wpre, wsc, wpost, out_ref):
    h = _merge_h(agg_ref, wpre)
    lane = lax.broadcasted_iota(jnp.int32, h.shape, 1)
    h = jnp.where(lane < 1, jnp.tanh(h), _gelu(h))
    out_ref[...] = (jnp.dot(x_ref[...], wsc[...], preferred_element_type=F32)
                    + jnp.dot(h, wpost[...], preferred_element_type=F32))


def _final_epilogue(agg, x, wpre, wsc, wpost):
    return pl.pallas_call(
        _final_body,
        grid=(N // _NB,),
        in_specs=[
            pl.BlockSpec((6, _NB, Q), lambda g: (0, g, 0)),
            pl.BlockSpec((_NB, C), lambda g: (g, 0)),
            pl.BlockSpec((2 * C, 8), lambda g: (0, 0)),
            pl.BlockSpec((C, 8), lambda g: (0, 0)),
            pl.BlockSpec((8, 8), lambda g: (0, 0)),
        ],
        out_specs=pl.BlockSpec((_NB, 8), lambda g: (g, 0)),
        out_shape=jax.ShapeDtypeStruct((N, 8), F32),
    )(agg, x, wpre, wsc, wpost)


def kernel(positions, senders, receivers, we1_1, we2_1, wsc_1, wpre_1, wpost_1,
           we1_2, we2_2, wsc_2, wpre_2, wpost_2, we1_3, we2_3, wsc_3, wpre_3,
           wpost_3):
    px = positions[:, 0]
    py = positions[:, 1]
    pz = positions[:, 2]
    z640 = jnp.zeros((640,), F32)
    z64 = jnp.zeros((B, Q), F32)

    shst, degf = _prep(px, py, pz, senders, receivers, z640)
    sh4 = shst.reshape(NCHUNK, 16, B)
    s1e, s2st = _s12_compute(sh4, we2_1, we2_2)
    t1f = _scal_scatter(s1e.reshape(E), receivers, z640)
    s3st = _s3_compute(sh4, we2_3)

    tab2, x2 = _l1_epilogue(degf.reshape(2, N, 1), t1f.reshape(2, N, 1),
                            we1_1, wpre_1, wpost_1, wsc_1, we1_2)

    agg2 = _agg_layer(tab2.reshape(6 * N, Q), s2st.reshape(3 * E, Q),
                      senders, receivers, z64)
    tab3, x3 = _mid_epilogue(agg2.reshape(6, N, Q), x2, wpre_2,
                             wsc_2, wpost_2, we1_3)

    agg3 = _agg_layer(tab3.reshape(6 * N, Q), s3st.reshape(3 * E, Q),
                      senders, receivers, z64)
    return _final_epilogue(agg3.reshape(6, N, Q), x3, wpre_3,
                           wsc_3, wpost_3)


# speedup vs baseline: 1.0247x; 1.0247x over previous
import math

import jax
import jax.numpy as jnp
from jax import lax
from jax.experimental import pallas as pl
from jax.experimental.pallas import tpu as pltpu
from jax.experimental.pallas import tpu_sc as plsc

N = 10000
E = 160000
C = 192
Q = 64
B = 128
NCHUNK = E // B
NS = 16
NC = 2
INV_SQRT2 = 0.7071067811865476
F32 = jnp.float32

_mesh = plsc.VectorSubcoreMesh(core_axis_name="c", subcore_axis_name="s")


def _prep_body(px, py, pz, send, recv, z640, shst, degf, *sc):
    (si0, ri0, si1, ri1, si2, ri2,
     c00, c01, c02, c03, c04, c05,
     c10, c11, c12, c13, c14, c15,
     c20, c21, c22, c23, c24, c25,
     t0, t1, t2, onesb, zb, degs,
     i0, i1, i2, p0, p1, p2, w0, w1, w2) = sc
    sidx = (si0, si1, si2)
    ridx = (ri0, ri1, ri2)
    comp = ((c00, c01, c02, c03, c04, c05),
            (c10, c11, c12, c13, c14, c15),
            (c20, c21, c22, c23, c24, c25))
    shT = (t0, t1, t2)
    semI = (i0, i1, i2)
    semP = (p0, p1, p2)
    semW = (w0, w1, w2)
    c = lax.axis_index("c")
    s = lax.axis_index("s")
    wid = s * NC + c

    pltpu.sync_copy(z640, zb)
    o640 = pl.multiple_of(640 * s, 128)

    @pl.when(s < 15)
    def _zmain():
        pltpu.sync_copy(zb, degs.at[pl.ds(o640, 640)])

    @pl.when(s == 15)
    def _ztail():
        pltpu.sync_copy(zb.at[pl.ds(0, 400)], degs.at[pl.ds(9600, 400)])

    for g in range(B // 16):
        onesb[pl.ds(g * 16, 16)] = jnp.full((16,), 1.0, F32)
    plsc.subcore_barrier()

    def e0_of(i):
        return (wid + 32 * i) * B

    def fire_idx(i, k):
        e0 = e0_of(i)
        pltpu.async_copy(send.at[pl.ds(e0, B)], sidx[k], semI[k])
        pltpu.async_copy(recv.at[pl.ds(e0, B)], ridx[k], semI[k])

    def wait_idx(i, k):
        e0 = e0_of(i)
        pltpu.make_async_copy(send.at[pl.ds(e0, B)], sidx[k], semI[k]).wait()
        pltpu.make_async_copy(recv.at[pl.ds(e0, B)], ridx[k], semI[k]).wait()

    def fire_gathers(k):
        cb = comp[k]
        pltpu.async_copy(px.at[sidx[k]], cb[0], semP[k])
        pltpu.async_copy(py.at[sidx[k]], cb[1], semP[k])
        pltpu.async_copy(pz.at[sidx[k]], cb[2], semP[k])
        pltpu.async_copy(px.at[ridx[k]], cb[3], semP[k])
        pltpu.async_copy(py.at[ridx[k]], cb[4], semP[k])
        pltpu.async_copy(pz.at[ridx[k]], cb[5], semP[k])

    def wait_gathers(k):
        cb = comp[k]
        pltpu.make_async_copy(px.at[sidx[k]], cb[0], semP[k]).wait()
        pltpu.make_async_copy(py.at[sidx[k]], cb[1], semP[k]).wait()
        pltpu.make_async_copy(pz.at[sidx[k]], cb[2], semP[k]).wait()
        pltpu.make_async_copy(px.at[ridx[k]], cb[3], semP[k]).wait()
        pltpu.make_async_copy(py.at[ridx[k]], cb[4], semP[k]).wait()
        pltpu.make_async_copy(pz.at[ridx[k]], cb[5], semP[k]).wait()

    def compute(i, k):
        wait_gathers(k)
        pltpu.make_async_copy(shT[k], shst.at[pl.ds(0, 16 * B)], semW[k]).wait()
        sxb, syb, szb, rxb, ryb, rzb = comp[k]
        tk = shT[k]
        s3 = math.sqrt(3.0)
        s5 = math.sqrt(5.0)
        s15 = math.sqrt(15.0)
        c1 = 0.25 * math.sqrt(70.0)
        c2 = math.sqrt(105.0)
        c3 = 0.25 * math.sqrt(42.0)
        c4 = 0.5 * math.sqrt(7.0)
        c5 = 0.5 * math.sqrt(105.0)
        for g in range(B // 16):
            sl = pl.ds(g * 16, 16)
            x = rxb[sl] - sxb[sl]
            y = ryb[sl] - syb[sl]
            z = rzb[sl] - szb[sl]
            x2 = x * x
            y2 = y * y
            z2c = z * z
            n2 = x2 + y2 + z2c
            shs = [
                s3 * x, s3 * y, s3 * z,
                s15 * x * y, s15 * y * z, 0.5 * s5 * (3.0 * z2c - n2),
                s15 * x * z, 0.5 * s15 * (x2 - y2),
                c1 * y * (3.0 * x2 - y2), c2 * x * y * z,
                c3 * y * (5.0 * z2c - n2), c4 * z * (5.0 * z2c - 3.0 * n2),
                c3 * x * (5.0 * z2c - n2), c5 * z * (x2 - y2),
                c1 * x * (x2 - 3.0 * y2),
                n2,
            ]
            for k2 in range(16):
                tk[pl.ds(k2 * B + g * 16, 16)] = shs[k2]
        e0 = e0_of(i)
        pltpu.async_copy(tk, shst.at[pl.ds(e0 * 16, 16 * B)], semW[k])
        pltpu.sync_copy(onesb, degs.at[ridx[k]], add=True)

    fire_idx(0, 0)
    fire_idx(1, 1)
    for k in range(3):
        pltpu.async_copy(shT[k], shst.at[pl.ds(e0_of(k) * 16, 16 * B)],
                         semW[k])
    wait_idx(0, 0)
    fire_gathers(0)
    fire_idx(2, 2)
    wait_idx(1, 1)
    fire_gathers(1)
    compute(0, 0)
    fire_idx(3, 0)
    wait_idx(2, 2)
    fire_gathers(2)
    compute(1, 1)
    fire_idx(4, 1)

    def outer2(jj, carry):
        for k in range(3):
            i = 3 + 3 * jj + k
            m = (k + 2) % 3
            wait_idx(i, k)
            fire_gathers(k)
            compute(i - 1, m)

            @pl.when(i + 2 < 39)
            def _fi():
                fire_idx(i + 2, m)

        return carry

    lax.fori_loop(0, 12, outer2, 0)
    compute(38, 38 % 3)
    for k in range(3):
        pltpu.make_async_copy(shT[k], shst.at[pl.ds(0, 16 * B)], semW[k]).wait()

    @pl.when(wid < 2)
    def _tailc():
        e0 = (1248 + wid) * B
        pltpu.async_copy(send.at[pl.ds(e0, B)], sidx[0], semI[0])
        pltpu.async_copy(recv.at[pl.ds(e0, B)], ridx[0], semI[0])
        pltpu.make_async_copy(send.at[pl.ds(e0, B)], sidx[0], semI[0]).wait()
        pltpu.make_async_copy(recv.at[pl.ds(e0, B)], ridx[0], semI[0]).wait()
        fire_gathers(0)
        wait_gathers(0)
        sxb, syb, szb, rxb, ryb, rzb = comp[0]
        tk = shT[0]
        s3 = math.sqrt(3.0)
        s5 = math.sqrt(5.0)
        s15 = math.sqrt(15.0)
        c1 = 0.25 * math.sqrt(70.0)
        c2 = math.sqrt(105.0)
        c3 = 0.25 * math.sqrt(42.0)
        c4 = 0.5 * math.sqrt(7.0)
        c5 = 0.5 * math.sqrt(105.0)
        for g in range(B // 16):
            sl = pl.ds(g * 16, 16)
            x = rxb[sl] - sxb[sl]
            y = ryb[sl] - syb[sl]
            z = rzb[sl] - szb[sl]
            x2 = x * x
            y2 = y * y
            z2c = z * z
            n2 = x2 + y2 + z2c
            shs = [
                s3 * x, s3 * y, s3 * z,
                s15 * x * y, s15 * y * z, 0.5 * s5 * (3.0 * z2c - n2),
                s15 * x * z, 0.5 * s15 * (x2 - y2),
                c1 * y * (3.0 * x2 - y2), c2 * x * y * z,
                c3 * y * (5.0 * z2c - n2), c4 * z * (5.0 * z2c - 3.0 * n2),
                c3 * x * (5.0 * z2c - n2), c5 * z * (x2 - y2),
                c1 * x * (x2 - 3.0 * y2),
                n2,
            ]
            for k2 in range(16):
                tk[pl.ds(k2 * B + g * 16, 16)] = shs[k2]
        pltpu.sync_copy(tk, shst.at[pl.ds(e0 * 16, 16 * B)])
        pltpu.sync_copy(onesb, degs.at[ridx[0]], add=True)

    plsc.subcore_barrier()

    @pl.when(s < 15)
    def _wmain():
        pltpu.sync_copy(degs.at[pl.ds(o640, 640)], zb)
        pltpu.sync_copy(zb, degf.at[pl.ds(c * N + 640 * s, 640)])

    @pl.when(s == 15)
    def _wtail():
        pltpu.sync_copy(degs.at[pl.ds(9600, 400)], zb.at[pl.ds(0, 400)])
        pltpu.sync_copy(zb.at[pl.ds(0, 400)], degf.at[pl.ds(c * N + 9600, 400)])


_prep = pl.kernel(
    _prep_body,
    out_type=(
        jax.ShapeDtypeStruct((NCHUNK * 16 * B,), F32),
        jax.ShapeDtypeStruct((2 * N,), F32),
    ),
    mesh=_mesh,
    scratch_types=(
        [pltpu.VMEM((B,), jnp.int32) for _ in range(6)]
        + [pltpu.VMEM((B,), F32) for _ in range(18)]
        + [pltpu.VMEM((16 * B,), F32) for _ in range(3)]
        + [pltpu.VMEM((B,), F32), pltpu.VMEM((640,), F32),
           pltpu.VMEM_SHARED((N,), F32)]
        + [pltpu.SemaphoreType.DMA for _ in range(9)]
    ),
)


def _scal_body(s1e, recv, z640, t1f, ridx, s1b, zb, tacc, sem):
    c = lax.axis_index("c")
    s = lax.axis_index("s")
    wid = s * NC + c

    pltpu.sync_copy(z640, zb)
    o640 = pl.multiple_of(640 * s, 128)

    @pl.when(s < 15)
    def _zmain():
        pltpu.sync_copy(zb, tacc.at[pl.ds(o640, 640)])

    @pl.when(s == 15)
    def _ztail():
        pltpu.sync_copy(zb.at[pl.ds(0, 400)], tacc.at[pl.ds(9600, 400)])

    plsc.subcore_barrier()
    nloc = 39 + jnp.where(wid < 2, 1, 0)

    def chunk_body(i, carry):
        chunk = wid + 32 * i
        e0 = chunk * B
        d1 = pltpu.async_copy(recv.at[pl.ds(e0, B)], ridx, sem)
        d2 = pltpu.async_copy(s1e.at[pl.ds(e0, B)], s1b, sem)
        d1.wait()
        d2.wait()
        pltpu.sync_copy(s1b, tacc.at[ridx], add=True)
        return carry

    lax.fori_loop(0, nloc, chunk_body, 0)
    plsc.subcore_barrier()

    @pl.when(s < 15)
    def _wmain():
        pltpu.sync_copy(tacc.at[pl.ds(o640, 640)], zb)
        pltpu.sync_copy(zb, t1f.at[pl.ds(c * N + 640 * s, 640)])

    @pl.when(s == 15)
    def _wtail():
        pltpu.sync_copy(tacc.at[pl.ds(9600, 400)], zb.at[pl.ds(0, 400)])
        pltpu.sync_copy(zb.at[pl.ds(0, 400)], t1f.at[pl.ds(c * N + 9600, 400)])


_scal_scatter = pl.kernel(
    _scal_body,
    out_type=jax.ShapeDtypeStruct((2 * N,), F32),
    mesh=_mesh,
    scratch_types=(
        pltpu.VMEM((B,), jnp.int32),
        pltpu.VMEM((B,), F32),
        pltpu.VMEM((640,), F32),
        pltpu.VMEM_SHARED((N,), F32),
        pltpu.SemaphoreType.DMA,
    ),
)


def _zero_acc(zb, aggs, s):
    r0 = pl.multiple_of(632 * s, 8)

    @pl.when(s < 15)
    def _zmain():
        for i in range(4):
            pltpu.sync_copy(zb, aggs.at[pl.ds(r0 + i * 128, 128)])
        pltpu.sync_copy(zb.at[pl.ds(0, 120)], aggs.at[pl.ds(r0 + 512, 120)])

    @pl.when(s == 15)
    def _ztail():
        for i in range(4):
            pltpu.sync_copy(zb, aggs.at[pl.ds(9480 + i * 128, 128)])
        pltpu.sync_copy(zb.at[pl.ds(0, 8)], aggs.at[pl.ds(9992, 8)])


def _write_acc(zb, aggs, aggf, s, ob):
    r0 = pl.multiple_of(632 * s, 8)

    @pl.when(s < 15)
    def _wmain():
        for i in range(4):
            pltpu.sync_copy(aggs.at[pl.ds(r0 + i * 128, 128)], zb)
            pltpu.sync_copy(zb, aggf.at[pl.ds(ob + 632 * s + i * 128, 128)])
        pltpu.sync_copy(aggs.at[pl.ds(r0 + 512, 120)], zb.at[pl.ds(0, 120)])
        pltpu.sync_copy(zb.at[pl.ds(0, 120)],
                        aggf.at[pl.ds(ob + 632 * s + 512, 120)])

    @pl.when(s == 15)
    def _wtail():
        for i in range(4):
            pltpu.sync_copy(aggs.at[pl.ds(9480 + i * 128, 128)], zb)
            pltpu.sync_copy(zb, aggf.at[pl.ds(ob + 9480 + i * 128, 128)])
        pltpu.sync_copy(aggs.at[pl.ds(9992, 8)], zb.at[pl.ds(0, 8)])
        pltpu.sync_copy(zb.at[pl.ds(0, 8)], aggf.at[pl.ds(ob + 9992, 8)])


def _agg_body(tab, s_st, send, recv, z64, aggf, *sc):
    (is0, ir0, is1, ir1, is2, ir2, is3, ir3, is4, ir4, is5, ir5,
     g0, sb0, g1, sb1, g2, sb2, zb, aggs,
     i0, i1, i2, i3, i4, i5,
     sg0, sg1, sg2, ss0, ss1, ss2, sb0s, sb1s, sb2s) = sc
    isx = (is0, is1, is2, is3, is4, is5)
    irx = (ir0, ir1, ir2, ir3, ir4, ir5)
    gbuf = (g0, g1, g2)
    sbuf = (sb0, sb1, sb2)
    semI = (i0, i1, i2, i3, i4, i5)
    semG = (sg0, sg1, sg2)
    semS = (ss0, ss1, ss2)
    semB = (sb0s, sb1s, sb2s)
    c = lax.axis_index("c")
    s = lax.axis_index("s")

    def run_pass(p):
        smode = (0, 1, 2)[p]
        j = 2 * p + c
        off = j * N
        soff = (j - 3) * E
        pltpu.sync_copy(z64, zb)
        _zero_acc(zb, aggs, s)
        plsc.subcore_barrier()

        def e0_of(i):
            return (s + NS * i) * B

        def fire_idx(i, q):
            e0 = e0_of(i)
            pltpu.async_copy(send.at[pl.ds(e0, B)], isx[q], semI[q])
            pltpu.async_copy(recv.at[pl.ds(e0, B)], irx[q], semI[q])

        def wait_idx(i, q):
            e0 = e0_of(i)
            pltpu.make_async_copy(send.at[pl.ds(e0, B)], isx[q], semI[q]).wait()
            pltpu.make_async_copy(recv.at[pl.ds(e0, B)], irx[q], semI[q]).wait()

        def fire_sb(b, e0):
            if smode == 0:
                return
            if smode == 1:
                @pl.when(c == 1)
                def _fs():
                    pltpu.async_copy(s_st.at[pl.ds(soff + e0, B)], sbuf[b],
                                     semB[b])
            else:
                pltpu.async_copy(s_st.at[pl.ds(soff + e0, B)], sbuf[b],
                                 semB[b])

        def mult(b):
            gb = gbuf[b]
            sb = sbuf[b]

            def row_body(r4, rc):
                for u in range(4):
                    r = r4 * 4 + u
                    for k in range(Q // 16):
                        gb[r, pl.ds(16 * k, 16)] = (gb[r, pl.ds(16 * k, 16)]
                                                    * sb[r, pl.ds(16 * k, 16)])
                return rc

            lax.fori_loop(0, B // 4, row_body, 0)

        def wait_mult(b):
            if smode == 0:
                return
            if smode == 1:
                @pl.when(c == 1)
                def _m():
                    pltpu.make_async_copy(s_st.at[pl.ds(0, B)], sbuf[b],
                                          semB[b]).wait()
                    mult(b)
            else:
                pltpu.make_async_copy(s_st.at[pl.ds(0, B)], sbuf[b],
                                      semB[b]).wait()
                mult(b)

        def fire_gather(i, b, q):
            wait_idx(i, q)
            for g in range(B // 16):
                sl = pl.ds(g * 16, 16)
                isx[q][sl] = isx[q][sl] + off
            pltpu.async_copy(tab.at[isx[q]], gbuf[b], semG[b])
            fire_sb(b, e0_of(i))

        def fire_b(b, q):
            pltpu.make_async_copy(tab.at[isx[q]], gbuf[b], semG[b]).wait()
            wait_mult(b)
            pltpu.async_copy(gbuf[b], aggs.at[irx[q]], semS[b], add=True)

        def wait_s(b, q):
            pltpu.make_async_copy(gbuf[b], aggs.at[irx[q]], semS[b]).wait()

        fire_idx(0, 0)
        fire_idx(1, 1)
        fire_idx(2, 2)

        def outer(jj, carry):
            for k in range(6):
                i = 6 * jj + k
                b = k % 3

                @pl.when(i >= 3)
                def _ws():
                    wait_s(b, (k + 3) % 6)

                @pl.when(i + 3 < 78)
                def _fi():
                    fire_idx(i + 3, (k + 3) % 6)

                fire_gather(i, b, k)

                @pl.when(i >= 2)
                def _fb():
                    fire_b((b + 1) % 3, (k + 4) % 6)

            return carry

        lax.fori_loop(0, 13, outer, 0)
        fire_b(1, 4)
        fire_b(2, 5)
        wait_s(0, 3)
        wait_s(1, 4)
        wait_s(2, 5)

        @pl.when(s < 2)
        def _tail():
            e0 = (1248 + s) * B
            d1 = pltpu.async_copy(send.at[pl.ds(e0, B)], isx[0], semI[0])
            d2 = pltpu.async_copy(recv.at[pl.ds(e0, B)], irx[0], semI[0])
            fire_sb(0, e0)
            d1.wait()
            d2.wait()
            for g in range(B // 16):
                sl = pl.ds(g * 16, 16)
                isx[0][sl] = isx[0][sl] + off
            pltpu.async_copy(tab.at[isx[0]], gbuf[0], semG[0]).wait()
            if smode == 1:
                @pl.when(c == 1)
                def _m():
                    pltpu.make_async_copy(s_st.at[pl.ds(0, B)], sbuf[0],
                                          semB[0]).wait()
                    mult(0)
            elif smode == 2:
                pltpu.make_async_copy(s_st.at[pl.ds(0, B)], sbuf[0],
                                      semB[0]).wait()
                mult(0)
            pltpu.sync_copy(gbuf[0], aggs.at[irx[0]], add=True)

        plsc.subcore_barrier()
        _write_acc(zb, aggs, aggf, s, j * N)
        plsc.subcore_barrier()

    run_pass(0)
    run_pass(1)
    run_pass(2)


_agg_layer = pl.kernel(
    _agg_body,
    out_type=jax.ShapeDtypeStruct((6 * N, Q), F32),
    mesh=_mesh,
    compiler_params=pltpu.CompilerParams(use_tc_tiling_on_sc=False),
    scratch_types=(
        [pltpu.VMEM((B,), jnp.int32) for _ in range(12)]
        + [pltpu.VMEM((B, Q), F32) for _ in range(6)]
        + [pltpu.VMEM((B, Q), F32), pltpu.VMEM_SHARED((N, Q), F32)]
        + [pltpu.SemaphoreType.DMA for _ in range(15)]
    ),
)


def _gelu(x):
    return 0.5 * x * (1.0 + jnp.tanh(0.7978845608028654 * (x + 0.044715 * x * x * x)))


def _act12(h):
    lane = lax.broadcasted_iota(jnp.int32, h.shape, 1)
    return jnp.where(lane < 32, _gelu(h), jnp.where(lane < 64, jnp.tanh(h), h))


_CB = 25
_C0 = ((0,), (0,)), ((), ())


def _s_norm(blk):
    n2 = blk[15:16, :]
    r = jnp.sqrt(n2)
    rinv = 1.0 / (r + 1e-8)
    rinv2 = rinv * rinv
    rinv3 = rinv2 * rinv
    fac = jnp.concatenate([
        jnp.broadcast_to(rinv, (3, B)),
        jnp.broadcast_to(rinv2, (5, B)),
        jnp.broadcast_to(rinv3, (7, B)),
    ], axis=0)
    shn = blk[:15, :] * fac
    fix5 = jnp.where(n2 == 0.0, -0.5 * math.sqrt(5.0), shn[5:6, :])
    return jnp.concatenate([shn[:5], fix5, shn[6:]], axis=0)


def _s12_body(sh_ref, w21_ref, w22_ref, w23_ref, s1_ref, s2_ref, s3_ref):
    w21 = w21_ref[...]
    w22 = w22_ref[...]
    w23 = w23_ref[...]
    for j in range(_CB):
        shn = _s_norm(sh_ref[j])
        sl = pl.ds(j * B, B)
        s1_ref[0, j, :] = lax.dot_general(w21, shn, _C0,
                                          preferred_element_type=F32).reshape(B)
        r2 = lax.dot_general(shn, w22, _C0, preferred_element_type=F32)
        r3 = lax.dot_general(shn, w23, _C0, preferred_element_type=F32)
        for t in range(3):
            s2_ref[t, sl, :] = r2[:, t * Q:(t + 1) * Q]
            s3_ref[t, sl, :] = r3[:, t * Q:(t + 1) * Q]


def _s12_compute(shst, w21, w22, w23):
    return pl.pallas_call(
        _s12_body,
        grid=(NCHUNK // _CB,),
        in_specs=[
            pl.BlockSpec((_CB, 16, B), lambda g: (g, 0, 0)),
            pl.BlockSpec((15, 1), lambda g: (0, 0)),
            pl.BlockSpec((15, C), lambda g: (0, 0)),
            pl.BlockSpec((15, C), lambda g: (0, 0)),
        ],
        out_specs=[
            pl.BlockSpec((1, _CB, B), lambda g: (g, 0, 0)),
            pl.BlockSpec((3, _CB * B, Q), lambda g: (0, g, 0)),
            pl.BlockSpec((3, _CB * B, Q), lambda g: (0, g, 0)),
        ],
        out_shape=[
            jax.ShapeDtypeStruct((NCHUNK // _CB, _CB, B), F32),
            jax.ShapeDtypeStruct((3, E, Q), F32),
            jax.ShapeDtypeStruct((3, E, Q), F32),
        ],
    )(shst, w21, w22, w23)


def _s3_body(sh_ref, w23_ref, s3_ref):
    w23 = w23_ref[...]
    for j in range(_CB):
        shn = _s_norm(sh_ref[j])
        sl = pl.ds(j * B, B)
        r3 = lax.dot_general(shn, w23, _C0, preferred_element_type=F32)
        for t in range(3):
            s3_ref[t, sl, :] = r3[:, t * Q:(t + 1) * Q]


def _s3_compute(shst, w23):
    return pl.pallas_call(
        _s3_body,
        grid=(NCHUNK // _CB,),
        in_specs=[
            pl.BlockSpec((_CB, 16, B), lambda g: (g, 0, 0)),
            pl.BlockSpec((15, C), lambda g: (0, 0)),
        ],
        out_specs=pl.BlockSpec((3, _CB * B, Q), lambda g: (0, g, 0)),
        out_shape=jax.ShapeDtypeStruct((3, E, Q), F32),
    )(shst, w23)


_NB = 2000


def _tab_write(tab_ref, x, xb):
    for t in range(3):
        tab_ref[t] = x[:, t * Q:(t + 1) * Q]
        tab_ref[3 + t] = xb[:, t * Q:(t + 1) * Q]


def _l1_body(deg_ref, t1_ref, we11, wpre, wpost, wsc, we1n, tab_ref, x2_ref):
    a0 = deg_ref[0] + deg_ref[1]
    t0 = (t1_ref[0] + t1_ref[1]) * we11[...]
    acat = jnp.concatenate([a0, t0], axis=1) * INV_SQRT2
    h = jnp.dot(acat, wpre[...], preferred_element_type=F32)
    h = _act12(h)
    x2 = wsc[...] + jnp.dot(h, wpost[...], preferred_element_type=F32)
    xb = jnp.dot(x2, we1n[...], preferred_element_type=F32)
    _tab_write(tab_ref, x2, xb)
    x2_ref[...] = x2


def _l1_epilogue(deg2, t12, we11, wpre, wpost, wsc, we1n):
    return pl.pallas_call(
        _l1_body,
        grid=(N // _NB,),
        in_specs=[
            pl.BlockSpec((2, _NB, 1), lambda g: (0, g, 0)),
            pl.BlockSpec((2, _NB, 1), lambda g: (0, g, 0)),
            pl.BlockSpec((1, 1), lambda g: (0, 0)),
            pl.BlockSpec((2, C), lambda g: (0, 0)),
            pl.BlockSpec((C, C), lambda g: (0, 0)),
            pl.BlockSpec((1, C), lambda g: (0, 0)),
            pl.BlockSpec((C, C), lambda g: (0, 0)),
        ],
        out_specs=[
            pl.BlockSpec((6, _NB, Q), lambda g: (0, g, 0)),
            pl.BlockSpec((_NB, C), lambda g: (g, 0)),
        ],
        out_shape=[
            jax.ShapeDtypeStruct((6, N, Q), F32),
            jax.ShapeDtypeStruct((N, C), F32),
        ],
    )(deg2, t12, we11, wpre, wpost, wsc, we1n)


def _merge_h(agg_ref, wpre_ref):
    acat = jnp.concatenate([agg_ref[j] for j in range(6)], axis=1)
    return jnp.dot(acat, wpre_ref[...], preferred_element_type=F32) * INV_SQRT2


def _mid_body(agg_ref, x_ref, wpre, wsc, wpost, we1n, tab_ref, xn_ref):
    h = _act12(_merge_h(agg_ref, wpre))
    xn = (jnp.dot(x_ref[...], wsc[...], preferred_element_type=F32)
          + jnp.dot(h, wpost[...], preferred_element_type=F32))
    xb = jnp.dot(xn, we1n[...], preferred_element_type=F32)
    _tab_write(tab_ref, xn, xb)
    xn_ref[...] = xn


def _mid_epilogue(agg, x, wpre, wsc, wpost, we1n):
    return pl.pallas_call(
        _mid_body,
        grid=(N // _NB,),
        in_specs=[
            pl.BlockSpec((6, _NB, Q), lambda g: (0, g, 0)),
            pl.BlockSpec((_NB, C), lambda g: (g, 0)),
            pl.BlockSpec((2 * C, C), lambda g: (0, 0)),
            pl.BlockSpec((C, C), lambda g: (0, 0)),
            pl.BlockSpec((C, C), lambda g: (0, 0)),
            pl.BlockSpec((C, C), lambda g: (0, 0)),
        ],
        out_specs=[
            pl.BlockSpec((6, _NB, Q), lambda g: (0, g, 0)),
            pl.BlockSpec((_NB, C), lambda g: (g, 0)),
        ],
        out_shape=[
            jax.ShapeDtypeStruct((6, N, Q), F32),
            jax.ShapeDtypeStruct((N, C), F32),
        ],
    )(agg, x, wpre, wsc, wpost, we1n)


def _final_body(agg_ref, x_ref, wpre, wsc, wpost, out_ref):
    h = _merge_h(agg_ref, wpre)
    lane = lax.broadcasted_iota(jnp.int32, h.shape, 1)
    h = jnp.where(lane < 1, jnp.tanh(h), _gelu(h))
    out_ref[...] = (jnp.dot(x_ref[...], wsc[...], preferred_element_type=F32)
                    + jnp.dot(h, wpost[...], preferred_element_type=F32))


def _final_epilogue(agg, x, wpre, wsc, wpost):
    return pl.pallas_call(
        _final_body,
        grid=(N // _NB,),
        in_specs=[
            pl.BlockSpec((6, _NB, Q), lambda g: (0, g, 0)),
            pl.BlockSpec((_NB, C), lambda g: (g, 0)),
            pl.BlockSpec((2 * C, 8), lambda g: (0, 0)),
            pl.BlockSpec((C, 8), lambda g: (0, 0)),
            pl.BlockSpec((8, 8), lambda g: (0, 0)),
        ],
        out_specs=pl.BlockSpec((_NB, 8), lambda g: (g, 0)),
        out_shape=jax.ShapeDtypeStruct((N, 8), F32),
    )(agg, x, wpre, wsc, wpost)


def kernel(positions, senders, receivers, we1_1, we2_1, wsc_1, wpre_1, wpost_1,
           we1_2, we2_2, wsc_2, wpre_2, wpost_2, we1_3, we2_3, wsc_3, wpre_3,
           wpost_3):
    px = positions[:, 0]
    py = positions[:, 1]
    pz = positions[:, 2]
    z640 = jnp.zeros((640,), F32)
    z64 = jnp.zeros((B, Q), F32)

    shst, degf = _prep(px, py, pz, senders, receivers, z640)
    sh4 = shst.reshape(NCHUNK, 16, B)
    s1e, s2st, s3st = _s12_compute(sh4, we2_1, we2_2, we2_3)
    t1f = _scal_scatter(s1e.reshape(E), receivers, z640)

    tab2, x2 = _l1_epilogue(degf.reshape(2, N, 1), t1f.reshape(2, N, 1),
                            we1_1, wpre_1, wpost_1, wsc_1, we1_2)

    agg2 = _agg_layer(tab2.reshape(6 * N, Q), s2st.reshape(3 * E, Q),
                      senders, receivers, z64)
    tab3, x3 = _mid_epilogue(agg2.reshape(6, N, Q), x2, wpre_2,
                             wsc_2, wpost_2, we1_3)

    agg3 = _agg_layer(tab3.reshape(6 * N, Q), s3st.reshape(3 * E, Q),
                      senders, receivers, z64)
    return _final_epilogue(agg3.reshape(6, N, Q), x3, wpre_3,
                           wsc_3, wpost_3)
